# Initial kernel scaffold; baseline (speedup 1.0000x reference)
import jax
import jax.numpy as jnp
from jax.experimental import pallas as pl


def kernel(x, gate_wi, gate_bi, gate_wo, gate_bo, exp_wi, exp_bi, exp_wo, exp_bo):
    raise NotImplementedError("write your pallas kernel here")


# speedup vs baseline: 6.7123x; 6.7123x over previous
import functools

import jax
import jax.numpy as jnp
from jax.experimental import pallas as pl
from jax.experimental.pallas import tpu as pltpu


_SELU_ALPHA = 1.6732632423543772848170429916717
_SELU_SCALE = 1.0507009873554804934193349852946


def _selu(v):
    return _SELU_SCALE * jnp.where(v > 0, v, _SELU_ALPHA * (jnp.exp(v) - 1.0))


def _gate_body(x_ref, gwi_ref, gbi_ref, gwo_ref, gbo_ref, w_ref, flags_ref,
               *, n_experts, tile_b):
    x = x_ref[...]
    h = _selu(
        jnp.dot(x, gwi_ref[...], preferred_element_type=jnp.float32)
        + gbi_ref[...])
    logits = (jnp.dot(h, gwo_ref[...], preferred_element_type=jnp.float32)
              + gbo_ref[...])
    ta, hdim = logits.shape
    iota = jax.lax.broadcasted_iota(jnp.int32, (ta, hdim), 1)
    m1 = jnp.max(logits, axis=1, keepdims=True)
    i1 = jnp.min(jnp.where(logits == m1, iota, hdim), axis=1, keepdims=True)
    neg = jnp.finfo(jnp.float32).min
    masked = jnp.where(iota == i1, neg, logits)
    m2 = jnp.max(masked, axis=1, keepdims=True)
    i2 = jnp.min(jnp.where(masked == m2, iota, hdim), axis=1, keepdims=True)
    s = m1 + m2
    p1 = m1 / s
    p2 = m2 / s
    eiota = jax.lax.broadcasted_iota(jnp.int32, (ta, n_experts), 1)
    hit1 = i1 == eiota
    hit2 = i2 == eiota
    w = (p1 * hit1.astype(jnp.float32) + p2 * hit2.astype(jnp.float32))
    w_ref[...] = w
    routed = (hit1 | hit2).astype(jnp.int32)
    nsub = ta // tile_b
    flags_ref[0] = jnp.max(routed.reshape(nsub, tile_b, n_experts), axis=1)


def _expert_body(flags_ref, x_ref, w_ref, wi_ref, bi_ref, wo_ref, bo_ref,
                 out_ref, *, n_tiles, tile_b, n_experts):
    e = pl.program_id(0)
    f = pl.program_id(1)

    @pl.when((e == 0) & (f == 0))
    def _init():
        out_ref[...] = jnp.zeros_like(out_ref)

    wi = wi_ref[0]
    wo = wo_ref[0]
    bi = bi_ref[0]
    bo = bo_ref[0]
    eiota = jax.lax.broadcasted_iota(jnp.int32, (1, n_experts), 1)
    bias_gate = (f == 0).astype(jnp.float32)

    def body(t, carry):
        @pl.when(flags_ref[t, e] > 0)
        def _tile():
            start = t * tile_b
            xs = x_ref[pl.ds(start, tile_b), :]
            wall = w_ref[pl.ds(start, tile_b), :]
            wcol = jnp.sum(wall * (eiota == e).astype(jnp.float32), axis=1,
                           keepdims=True)
            h = _selu(
                jnp.dot(xs, wi, preferred_element_type=jnp.float32) + bi)
            y = jnp.dot(h, wo, preferred_element_type=jnp.float32) * wcol
            y = y + (bo * wcol) * bias_gate
            out_ref[pl.ds(start, tile_b), :] += y
        return carry

    jax.lax.fori_loop(0, n_tiles, body, 0)


def kernel(x, gate_wi, gate_bi, gate_wo, gate_bo, exp_wi, exp_bi, exp_wo,
           exp_bo):
    b, s, hdim = x.shape
    n = b * s
    e_num, _, fdim = exp_wi.shape
    x_flat = x.reshape(n, hdim)

    tile_a = 256 if n % 256 == 0 else n
    tile_b = 64 if n % 64 == 0 else n
    n_tiles = n // tile_b
    nsub = tile_a // tile_b
    f_chunk = 1024 if fdim % 1024 == 0 else fdim
    nf = fdim // f_chunk

    gate = pl.pallas_call(
        functools.partial(_gate_body, n_experts=e_num, tile_b=tile_b),
        grid=(n // tile_a,),
        in_specs=[
            pl.BlockSpec((tile_a, hdim), lambda t: (t, 0)),
            pl.BlockSpec((hdim, fdim), lambda t: (0, 0)),
            pl.BlockSpec((1, fdim), lambda t: (0, 0)),
            pl.BlockSpec((fdim, hdim), lambda t: (0, 0)),
            pl.BlockSpec((1, hdim), lambda t: (0, 0)),
        ],
        out_specs=[
            pl.BlockSpec((tile_a, e_num), lambda t: (t, 0)),
            pl.BlockSpec((1, nsub, e_num), lambda t: (t, 0, 0)),
        ],
        out_shape=[
            jax.ShapeDtypeStruct((n, e_num), jnp.float32),
            jax.ShapeDtypeStruct((n // tile_a, nsub, e_num), jnp.int32),
        ],
    )
    w_tok, flags3 = gate(x_flat, gate_wi, gate_bi.reshape(1, fdim),
                         gate_wo, gate_bo.reshape(1, hdim))
    flags = flags3.reshape(n_tiles, e_num)

    expert = pl.pallas_call(
        functools.partial(_expert_body, n_tiles=n_tiles, tile_b=tile_b,
                          n_experts=e_num),
        grid=(e_num, nf),
        in_specs=[
            pl.BlockSpec(memory_space=pltpu.SMEM),
            pl.BlockSpec((n, hdim), lambda e, f: (0, 0)),
            pl.BlockSpec((n, e_num), lambda e, f: (0, 0)),
            pl.BlockSpec((1, hdim, f_chunk), lambda e, f: (e, 0, f)),
            pl.BlockSpec((1, 1, f_chunk), lambda e, f: (e, 0, f)),
            pl.BlockSpec((1, f_chunk, hdim), lambda e, f: (e, f, 0)),
            pl.BlockSpec((1, 1, hdim), lambda e, f: (e, 0, 0)),
        ],
        out_specs=pl.BlockSpec((n, hdim), lambda e, f: (0, 0)),
        out_shape=jax.ShapeDtypeStruct((n, hdim), jnp.float32),
    )
    out = expert(flags, x_flat, w_tok, exp_wi, exp_bi.reshape(e_num, 1, fdim),
                 exp_wo, exp_bo.reshape(e_num, 1, hdim))
    return out.reshape(b, s, hdim)
